# Initial kernel scaffold; baseline (speedup 1.0000x reference)
import jax
import jax.numpy as jnp
from jax.experimental import pallas as pl


def kernel(x, exe_ids, pe):
    raise NotImplementedError("write your pallas kernel here")


# speedup vs baseline: 1.9760x; 1.9760x over previous
import jax
import jax.numpy as jnp
from jax.experimental import pallas as pl


def _body(x_ref, e_ref, pe_ref, o_ref):
    scale = jnp.where(e_ref[...] != 0, 2.0, 1.0)
    o_ref[...] = x_ref[...] + pe_ref[...] * scale[:, :, None]


def kernel(x, exe_ids, pe):
    S, B, D = x.shape
    BS = 256
    grid = (S // BS,)
    return pl.pallas_call(
        _body,
        grid=grid,
        in_specs=[
            pl.BlockSpec((BS, B, D), lambda i: (i, 0, 0)),
            pl.BlockSpec((BS, B), lambda i: (i, 0)),
            pl.BlockSpec((BS, 1, D), lambda i: (i, 0, 0)),
        ],
        out_specs=pl.BlockSpec((BS, B, D), lambda i: (i, 0, 0)),
        out_shape=jax.ShapeDtypeStruct(x.shape, x.dtype),
    )(x, exe_ids, pe)
